# Initial kernel scaffold; baseline (speedup 1.0000x reference)
import jax
import jax.numpy as jnp
from jax.experimental import pallas as pl


def kernel(x, edge_index, edge_attr, node_emb, edge_W0, edge_b0, W1_0, b1_0, gamma0, beta0, W2_0, b2_0, edge_W1, edge_b1, W1_1, b1_1, gamma1, beta1, W2_1, b2_1):
    raise NotImplementedError("write your pallas kernel here")


# speedup vs baseline: 3.8233x; 3.8233x over previous
import functools

import jax
import jax.numpy as jnp
from jax import lax
from jax.experimental import pallas as pl
from jax.experimental.pallas import tpu as pltpu
from jax.experimental.pallas import tpu_sc as plsc

NC = 2
NS = 16
NW = NC * NS
CHUNK = 128


def _sc_scatter(feat, src3, dst3, zeros, erows=None):
  n_tab, d = zeros.shape
  _, nch, _ = src3.shape
  rpt = n_tab // NS

  scratch = [
      pltpu.VMEM((nch, CHUNK), jnp.int32),
      pltpu.VMEM((nch, CHUNK), jnp.int32),
      pltpu.VMEM((CHUNK, d), jnp.float32),
      pltpu.VMEM((CHUNK, d), jnp.float32),
      pltpu.VMEM_SHARED((n_tab, d), jnp.float32),
      pltpu.SemaphoreType.DMA,
  ]
  mesh = plsc.VectorSubcoreMesh(core_axis_name="c", subcore_axis_name="s")
  out_type = jax.ShapeDtypeStruct((NC, n_tab, d), jnp.float32)

  def body(*refs):
    if erows is None:
      (feat_hbm, src_hbm, dst_hbm, zeros_hbm, out_hbm,
       src_v, dst_v, rows_v, erow_v, table, sem) = refs
      erows_hbm = None
    else:
      (feat_hbm, src_hbm, dst_hbm, erows_hbm, zeros_hbm, out_hbm,
       src_v, dst_v, rows_v, erow_v, table, sem) = refs
    cid = lax.axis_index("c")
    sid = lax.axis_index("s")
    gw = cid * NS + sid
    pltpu.sync_copy(zeros_hbm.at[pl.ds(sid * rpt, rpt)],
                    table.at[pl.ds(sid * rpt, rpt)])
    pltpu.sync_copy(src_hbm.at[gw], src_v)
    pltpu.sync_copy(dst_hbm.at[gw], dst_v)
    plsc.subcore_barrier()

    def step(j, carry):
      pltpu.async_copy(feat_hbm.at[src_v.at[j]], rows_v, sem).wait()
      pltpu.sync_copy(rows_v, table.at[dst_v.at[j]], add=True)
      if erows is not None:
        pltpu.sync_copy(erows_hbm.at[gw, pl.ds(j * CHUNK, CHUNK)], erow_v)
        pltpu.sync_copy(erow_v, table.at[dst_v.at[j]], add=True)
      return carry

    lax.fori_loop(0, nch, step, 0)
    plsc.subcore_barrier()
    pltpu.sync_copy(table.at[pl.ds(sid * rpt, rpt)],
                    out_hbm.at[cid, pl.ds(sid * rpt, rpt)])

  args = [feat, src3, dst3]
  if erows is not None:
    args.append(erows)
  args.append(zeros)
  fn = pl.kernel(
      body, out_type=out_type, mesh=mesh, scratch_types=scratch,
      compiler_params=pltpu.CompilerParams(use_tc_tiling_on_sc=False))
  return fn(*args)


def _dense0(S, corr, node_emb, ew, eb, w1, b1, g, bt, w2, b2, n, d):
  def body(s_ref, corr_ref, ne_ref, ew_ref, eb_ref, w1_ref, b1_ref, g_ref,
           bt_ref, w2_ref, b2_ref, h1_ref, sc_ref):
    Sc = s_ref[0, :n, :] + s_ref[1, :n, :] + corr_ref[...]
    sc_ref[...] = Sc
    emb0 = ne_ref[0:1, :]
    demb = ne_ref[1:2, :] - emb0
    deg = Sc[:, 0:1]
    cnt1 = Sc[:, 1:2]
    agg_h = deg * emb0 + cnt1 * demb
    ew = ew_ref[...].astype(jnp.bfloat16).astype(jnp.float32)
    agg_ee = (Sc[:, 2:3] * ew[0:1, :] + Sc[:, 3:4] * ew[1:2, :]
              + Sc[:, 4:5] * ew[2:3, :] + deg * eb_ref[...])
    aggr = jnp.concatenate([agg_h, agg_ee], axis=1)
    z = jnp.dot(aggr, w1_ref[...], preferred_element_type=jnp.float32) + b1_ref[...]
    m = jnp.mean(z, axis=0, keepdims=True)
    v = jnp.mean((z - m) * (z - m), axis=0, keepdims=True)
    z = (z - m) / jnp.sqrt(v + 1e-5) * g_ref[...] + bt_ref[...]
    z = jnp.maximum(z, 0.0)
    h1 = jnp.dot(z, w2_ref[...], preferred_element_type=jnp.float32) + b2_ref[...]
    h1_ref[...] = jnp.maximum(h1, 0.0)

  return pl.pallas_call(
      body,
      out_shape=[
          jax.ShapeDtypeStruct((n, d), jnp.float32),
          jax.ShapeDtypeStruct((n, 16), jnp.float32),
      ],
  )(S, corr, node_emb, ew, eb, w1, b1, g, bt, w2, b2)


def _dense1(G, h1, Sc, ew, eb, w1, b1, g, bt, w2, b2, n, d):
  n_tab = G.shape[1]
  bs = 2000
  nb = n // bs

  def body_a(g3_ref, h1_ref, sc_ref, ew_ref, eb_ref, w1_ref, b1_ref, z_ref):
    aggh = g3_ref[0] + g3_ref[1] + h1_ref[...]
    Sc = sc_ref[...]
    ew = ew_ref[...].astype(jnp.bfloat16).astype(jnp.float32)
    agg_ee = (Sc[:, 2:3] * ew[0:1, :] + Sc[:, 3:4] * ew[1:2, :]
              + Sc[:, 4:5] * ew[2:3, :] + Sc[:, 0:1] * eb_ref[...])
    aggr = jnp.concatenate([aggh, agg_ee], axis=1)
    z_ref[...] = jnp.dot(aggr, w1_ref[...], preferred_element_type=jnp.float32) + b1_ref[...]

  z = pl.pallas_call(
      body_a,
      grid=(nb,),
      in_specs=[
          pl.BlockSpec((NC, bs, d), lambda i: (0, i, 0)),
          pl.BlockSpec((bs, d), lambda i: (i, 0)),
          pl.BlockSpec((bs, 16), lambda i: (i, 0)),
          pl.BlockSpec((3, d), lambda i: (0, 0)),
          pl.BlockSpec((1, d), lambda i: (0, 0)),
          pl.BlockSpec((2 * d, 2 * d), lambda i: (0, 0)),
          pl.BlockSpec((1, 2 * d), lambda i: (0, 0)),
      ],
      out_specs=pl.BlockSpec((bs, 2 * d), lambda i: (i, 0)),
      out_shape=jax.ShapeDtypeStruct((n, 2 * d), jnp.float32),
  )(G, h1, Sc, ew, eb, w1, b1)

  def body_b(z_ref, gm_ref, bt_ref, w2_ref, b2_ref, out_ref):
    z = z_ref[...]
    m = jnp.mean(z, axis=0, keepdims=True)
    v = jnp.mean((z - m) * (z - m), axis=0, keepdims=True)
    z = (z - m) / jnp.sqrt(v + 1e-5) * gm_ref[...] + bt_ref[...]
    z = jnp.maximum(z, 0.0)
    out_ref[...] = (jnp.dot(z, w2_ref[...], preferred_element_type=jnp.float32)
                    + b2_ref[...])

  return pl.pallas_call(
      body_b,
      out_shape=jax.ShapeDtypeStruct((n, d), jnp.float32),
  )(z, g, bt, w2, b2)


def kernel(x, edge_index, edge_attr, node_emb, edge_W0, edge_b0, W1_0, b1_0,
           gamma0, beta0, W2_0, b2_0, edge_W1, edge_b1, W1_1, b1_1, gamma1,
           beta1, W2_1, b2_1):
  n = x.shape[0]
  e = edge_index.shape[1]
  d = node_emb.shape[1]
  xf = x.astype(jnp.float32)

  nch = -(-e // (NW * CHUNK))
  ep = NW * nch * CHUNK
  pad = ep - e
  src3 = jnp.concatenate(
      [edge_index[0], jnp.zeros((pad,), jnp.int32)]).reshape(NW, nch, CHUNK)
  dst3 = jnp.concatenate(
      [edge_index[1], jnp.full((pad,), n, jnp.int32)]).reshape(NW, nch, CHUNK)

  n_tab = -(-(n + 1) // (NS * 8)) * (NS * 8)

  t0 = jnp.zeros((n, 16), jnp.float32).at[:, 0].set(1.0).at[:, 1].set(xf)
  ea_r = edge_attr.astype(jnp.bfloat16).astype(jnp.float32)
  er3 = (jnp.zeros((ep, 16), jnp.float32).at[:e, 2:5].set(ea_r)
         .reshape(NW, nch * CHUNK, 16))
  zeros16 = jnp.zeros((n_tab, 16), jnp.float32)
  zerosd = jnp.zeros((n_tab, d), jnp.float32)
  corr = (jnp.zeros((n, 16), jnp.float32).at[:, 0].set(1.0)
          .at[:, 1].set(xf).at[:, 2].set(1.0))

  r1 = lambda a: a.reshape(1, -1)

  S = _sc_scatter(t0, src3, dst3, zeros16, erows=er3)
  h1, Sc = _dense0(S, corr, node_emb, edge_W0, r1(edge_b0), W1_0, r1(b1_0),
                   r1(gamma0), r1(beta0), W2_0, r1(b2_0), n, d)
  G = _sc_scatter(h1, src3, dst3, zerosd)
  out = _dense1(G, h1, Sc, edge_W1, r1(edge_b1), W1_1, r1(b1_1), r1(gamma1),
                r1(beta1), W2_1, r1(b2_1), n, d)
  return out
